# baseline (speedup 1.0000x reference)
import functools

import jax
import jax.numpy as jnp
from jax.experimental import pallas as pl
from jax.experimental.pallas import tpu as pltpu

_F = 32


def _fpmc_tc(L):
    def body(pk_ref, v_il, v_li, v_ul, v_lu, v_ui, v_iu, out_ref,
             rows_li, rows_lu, row_il, row_iu, row_ul, row_ui, sem):
        copies = []
        for l in range(L):
            idx = pk_ref[l] - 1
            copies.append(pltpu.make_async_copy(
                v_li.at[pl.ds(idx, 1)], rows_li.at[pl.ds(l, 1)], sem))
            copies.append(pltpu.make_async_copy(
                v_lu.at[pl.ds(idx, 1)], rows_lu.at[pl.ds(l, 1)], sem))
        i0 = pk_ref[56] - 1
        u0 = pk_ref[64] - 1
        copies.append(pltpu.make_async_copy(v_il.at[pl.ds(i0, 1)], row_il, sem))
        copies.append(pltpu.make_async_copy(v_iu.at[pl.ds(i0, 1)], row_iu, sem))
        copies.append(pltpu.make_async_copy(v_ul.at[pl.ds(u0, 1)], row_ul, sem))
        copies.append(pltpu.make_async_copy(v_ui.at[pl.ds(u0, 1)], row_ui, sem))
        for c in copies:
            c.start()
        for c in copies:
            c.wait()

        fac = jnp.where(pk_ref[72] > 0, jnp.float32(1.0 / L), jnp.float32(0.0))
        mc = (jnp.sum(rows_li[...] * row_il[...])
              + jnp.sum(rows_lu[...] * row_ul[...])) * fac
        mf = jnp.sum(row_ui[...] * row_iu[...])
        out_ref[0] = mc + mf

    grid_spec = pltpu.PrefetchScalarGridSpec(
        num_scalar_prefetch=1,
        grid=(),
        in_specs=[pl.BlockSpec(memory_space=pltpu.HBM)] * 6,
        out_specs=pl.BlockSpec(memory_space=pltpu.SMEM),
        scratch_shapes=[
            pltpu.VMEM((L, _F), jnp.float32),
            pltpu.VMEM((L, _F), jnp.float32),
            pltpu.VMEM((1, _F), jnp.float32),
            pltpu.VMEM((1, _F), jnp.float32),
            pltpu.VMEM((1, _F), jnp.float32),
            pltpu.VMEM((1, _F), jnp.float32),
            pltpu.SemaphoreType.DMA,
        ],
    )
    return pl.pallas_call(
        body,
        grid_spec=grid_spec,
        out_shape=jax.ShapeDtypeStruct((1,), jnp.float32),
        compiler_params=pltpu.CompilerParams(
            allow_input_fusion=(True,) * 7),
    )


def kernel(u, i, t, last_basket, V_IL, V_LI, V_UL, V_LU, V_UI, V_IU):
    L = last_basket.shape[0]
    lb = last_basket.astype(jnp.int32)
    packed = jnp.concatenate([
        lb,
        jnp.ones((56 - L,), jnp.int32),
        jnp.asarray(i, jnp.int32)[None],
        jnp.ones((7,), jnp.int32),
        jnp.asarray(u, jnp.int32)[None],
        jnp.ones((7,), jnp.int32),
        jnp.asarray(t, jnp.int32)[None],
        jnp.ones((7,), jnp.int32),
    ])
    out = _fpmc_tc(L)(packed, V_IL, V_LI, V_UL, V_LU, V_UI, V_IU)
    return out[0]


# speedup vs baseline: 107.0596x; 107.0596x over previous
import jax
import jax.numpy as jnp
from jax import lax
from jax.experimental import pallas as pl
from jax.experimental.pallas import tpu as pltpu

_F = 32
_TL = 128


def _fpmc_tc(L):
    def body(pk_ref, v_il, v_li, v_ul, v_lu, v_ui, v_iu, out_ref,
             blks_li, blks_lu, blk_il, blk_iu, blk_ul, blk_ui, sem):
        def tile_copy(src, dst, idx):
            base = pl.multiple_of((idx // _TL) * _TL, _TL)
            return pltpu.make_async_copy(
                src.at[:, pl.ds(base, _TL)], dst, sem)

        copies = []
        for l in range(L):
            idx = pk_ref[l] - 1
            copies.append(tile_copy(v_li, blks_li.at[l], idx))
            copies.append(tile_copy(v_lu, blks_lu.at[l], idx))
        i0 = pk_ref[56] - 1
        u0 = pk_ref[64] - 1
        copies.append(tile_copy(v_il, blk_il, i0))
        copies.append(tile_copy(v_iu, blk_iu, i0))
        copies.append(tile_copy(v_ul, blk_ul, u0))
        copies.append(tile_copy(v_ui, blk_ui, u0))
        for c in copies:
            c.start()
        for c in copies:
            c.wait()

        lane = lax.broadcasted_iota(jnp.int32, (_F, _TL), 1)

        def col(blk, idx):
            return jnp.sum(jnp.where(lane == idx % _TL, blk, 0.0), axis=1)

        acc_li = col(blks_li[0], pk_ref[0] - 1)
        acc_lu = col(blks_lu[0], pk_ref[0] - 1)
        for l in range(1, L):
            idx = pk_ref[l] - 1
            acc_li = acc_li + col(blks_li[l], idx)
            acc_lu = acc_lu + col(blks_lu[l], idx)

        vi = col(blk_il[...], i0)
        vu = col(blk_ul[...], u0)
        vui = col(blk_ui[...], u0)
        viu = col(blk_iu[...], i0)

        fac = jnp.where(pk_ref[72] > 0, jnp.float32(1.0 / L), jnp.float32(0.0))
        mc = (jnp.sum(acc_li * vi) + jnp.sum(acc_lu * vu)) * fac
        mf = jnp.sum(vui * viu)
        out_ref[0] = mc + mf

    grid_spec = pltpu.PrefetchScalarGridSpec(
        num_scalar_prefetch=1,
        grid=(),
        in_specs=[pl.BlockSpec(memory_space=pltpu.HBM)] * 6,
        out_specs=pl.BlockSpec(memory_space=pltpu.SMEM),
        scratch_shapes=[
            pltpu.VMEM((L, _F, _TL), jnp.float32),
            pltpu.VMEM((L, _F, _TL), jnp.float32),
            pltpu.VMEM((_F, _TL), jnp.float32),
            pltpu.VMEM((_F, _TL), jnp.float32),
            pltpu.VMEM((_F, _TL), jnp.float32),
            pltpu.VMEM((_F, _TL), jnp.float32),
            pltpu.SemaphoreType.DMA,
        ],
    )
    return pl.pallas_call(
        body,
        grid_spec=grid_spec,
        out_shape=jax.ShapeDtypeStruct((1,), jnp.float32),
    )


def kernel(u, i, t, last_basket, V_IL, V_LI, V_UL, V_LU, V_UI, V_IU):
    L = last_basket.shape[0]
    lb = last_basket.astype(jnp.int32)
    packed = jnp.concatenate([
        lb,
        jnp.ones((56 - L,), jnp.int32),
        jnp.asarray(i, jnp.int32)[None],
        jnp.ones((7,), jnp.int32),
        jnp.asarray(u, jnp.int32)[None],
        jnp.ones((7,), jnp.int32),
        jnp.asarray(t, jnp.int32)[None],
        jnp.ones((7,), jnp.int32),
    ])
    out = _fpmc_tc(L)(packed, V_IL.T, V_LI.T, V_UL.T, V_LU.T, V_UI.T, V_IU.T)
    return out[0]
